# baseline (speedup 1.0000x reference)
import functools

import jax
import jax.numpy as jnp
from jax import lax
from jax.experimental import pallas as pl
from jax.experimental.pallas import tpu as pltpu
from jax.experimental.pallas import tpu_sc as plsc

_NC, _NS = 2, 16
_NW = _NC * _NS
_H = 64
_B = 16384
_N = 64
_BPW = _B // _NW
_MARGIN = 1.0
_BLK = 2048


def _sc_gather(emb, r_emb, pos_h, pos_r, pos_t, neg_h, neg_t, neg_r):
    mesh = plsc.VectorSubcoreMesh(core_axis_name="c", subcore_axis_name="s")
    out_type = (
        [jax.ShapeDtypeStruct((_B, _H), jnp.float32)] * 3
        + [jax.ShapeDtypeStruct((_N, _H), jnp.float32)] * 3
    )
    scratch_types = [
        pltpu.VMEM((_BPW,), jnp.int32),
        pltpu.VMEM((_BPW,), jnp.int32),
        pltpu.VMEM((_BPW,), jnp.int32),
        pltpu.VMEM((_BPW, _H), jnp.float32),
        pltpu.VMEM((_BPW, _H), jnp.float32),
        pltpu.VMEM((_BPW, _H), jnp.float32),
        pltpu.VMEM((_N,), jnp.int32),
        pltpu.VMEM((_N, _H), jnp.float32),
        pltpu.SemaphoreType.DMA,
        pltpu.SemaphoreType.DMA,
        pltpu.SemaphoreType.DMA,
        pltpu.SemaphoreType.DMA,
        pltpu.SemaphoreType.DMA,
        pltpu.SemaphoreType.DMA,
    ]

    @functools.partial(pl.kernel, mesh=mesh, out_type=out_type,
                       scratch_types=scratch_types,
                       compiler_params=pltpu.CompilerParams(
                           use_tc_tiling_on_sc=False))
    def k(emb_hbm, remb_hbm, ph_hbm, pr_hbm, pt_hbm, nh_hbm, nt_hbm, nr_hbm,
          oh, orr, ot, onh, ont, onr,
          ih_v, ir_v, it_v, rh_v, rr_v, rt_v, ni_v, nrow_v,
          sem0, sem1, sem2, sem3, sem4, sem5):
        wid = lax.axis_index("s") * _NC + lax.axis_index("c")
        base = wid * _BPW
        sl = pl.ds(base, _BPW)
        ci_h = pltpu.async_copy(ph_hbm.at[sl], ih_v, sem0)
        ci_r = pltpu.async_copy(pr_hbm.at[sl], ir_v, sem1)
        ci_t = pltpu.async_copy(pt_hbm.at[sl], it_v, sem2)
        ci_h.wait()
        g_h = pltpu.async_copy(emb_hbm.at[ih_v], rh_v, sem0)
        ci_r.wait()
        g_r = pltpu.async_copy(remb_hbm.at[ir_v], rr_v, sem1)
        ci_t.wait()
        g_t = pltpu.async_copy(emb_hbm.at[it_v], rt_v, sem2)
        g_h.wait()
        w_h = pltpu.async_copy(rh_v, oh.at[sl], sem3)
        g_r.wait()
        w_r = pltpu.async_copy(rr_v, orr.at[sl], sem4)
        g_t.wait()
        w_t = pltpu.async_copy(rt_v, ot.at[sl], sem5)

        @pl.when(wid == 0)
        def _():
            pltpu.sync_copy(nh_hbm, ni_v)
            pltpu.async_copy(emb_hbm.at[ni_v], nrow_v, sem0).wait()
            pltpu.sync_copy(nrow_v, onh)

        @pl.when(wid == 1)
        def _():
            pltpu.sync_copy(nt_hbm, ni_v)
            pltpu.async_copy(emb_hbm.at[ni_v], nrow_v, sem0).wait()
            pltpu.sync_copy(nrow_v, ont)

        @pl.when(wid == 2)
        def _():
            pltpu.sync_copy(nr_hbm, ni_v)
            pltpu.async_copy(remb_hbm.at[ni_v], nrow_v, sem0).wait()
            pltpu.sync_copy(nrow_v, onr)

        w_h.wait()
        w_r.wait()
        w_t.wait()

    return k(emb, r_emb, pos_h, pos_r, pos_t, neg_h, neg_t, neg_r)


_LOG2E = 1.4426950408889634
_LN2 = 0.6931471805599453


def _softplus_neg_sum(x):
    return jnp.sum(jnp.log2(1.0 + jnp.exp2(x * _LOG2E)))


def _softplus_stable(x):
    return jnp.maximum(x, 0.0) + _LN2 * jnp.log2(
        1.0 + jnp.exp2(-jnp.abs(x) * _LOG2E))


def _tc_body(h_ref, r_ref, t_ref, nh_ref, nt_ref, nr_ref, o_ref):
    i = pl.program_id(0)

    @pl.when(i == 0)
    def _():
        o_ref[...] = jnp.zeros((1, 1), jnp.float32)

    h = h_ref[...]
    r = r_ref[...]
    t = t_ref[...]
    nh = nh_ref[...]
    nt = nt_ref[...]
    nr = nr_ref[...]

    d = h + r - t
    rt = r - t
    hr = h + r
    ht = h - t

    def dot_t(a, b):
        return lax.dot_general(a.astype(jnp.bfloat16), b.astype(jnp.bfloat16),
                               (((1,), (1,)), ((), ())),
                               preferred_element_type=jnp.float32)

    nh2 = 0.5 * jnp.sum(nh * nh, axis=1)
    nt2 = 0.5 * jnp.sum(nt * nt, axis=1)
    nr2 = 0.5 * jnp.sum(nr * nr, axis=1)

    pos = _MARGIN - 0.5 * jnp.sum(d * d, axis=1)
    pos_sum = jnp.sum(_softplus_stable(-pos))

    y_nh = (_MARGIN - 0.5 * jnp.sum(rt * rt, axis=1, keepdims=True)
            - nh2[None, :] - dot_t(rt, nh))
    y_nt = (_MARGIN - 0.5 * jnp.sum(hr * hr, axis=1, keepdims=True)
            - nt2[None, :] + dot_t(hr, nt))
    y_nr = (_MARGIN - 0.5 * jnp.sum(ht * ht, axis=1, keepdims=True)
            - nr2[None, :] - dot_t(ht, nr))

    part = (3.0 * pos_sum
            + _LN2 * (_softplus_neg_sum(y_nh)
                      + _softplus_neg_sum(y_nt)
                      + _softplus_neg_sum(y_nr)))
    o_ref[...] += jnp.full((1, 1), part * (1.0 / _B), jnp.float32)


def _tc_score(h_e, r_e, t_e, nh_e, nt_e, nr_e):
    grid = _B // _BLK
    out = pl.pallas_call(
        _tc_body,
        grid=(grid,),
        in_specs=(
            [pl.BlockSpec((_BLK, _H), lambda i: (i, 0))] * 3
            + [pl.BlockSpec((_N, _H), lambda i: (0, 0))] * 3
        ),
        out_specs=pl.BlockSpec((1, 1), lambda i: (0, 0)),
        out_shape=jax.ShapeDtypeStruct((1, 1), jnp.float32),
    )(h_e, r_e, t_e, nh_e, nt_e, nr_e)
    return out[0, 0]


def kernel(pos_h, pos_r, pos_t, neg_h, neg_t, neg_r, emb, r_emb):
    pos_h = pos_h.astype(jnp.int32)
    pos_r = pos_r.astype(jnp.int32)
    pos_t = pos_t.astype(jnp.int32)
    neg_h = neg_h.astype(jnp.int32)
    neg_t = neg_t.astype(jnp.int32)
    neg_r = neg_r.astype(jnp.int32)
    h_e, r_e, t_e, nh_e, nt_e, nr_e = _sc_gather(
        emb, r_emb, pos_h, pos_r, pos_t, neg_h, neg_t, neg_r)
    return _tc_score(h_e, r_e, t_e, nh_e, nt_e, nr_e)


# speedup vs baseline: 1.3978x; 1.3978x over previous
import functools

import jax
import jax.numpy as jnp
from jax import lax
from jax.experimental import pallas as pl
from jax.experimental.pallas import tpu as pltpu
from jax.experimental.pallas import tpu_sc as plsc

_NC, _NS = 2, 16
_NW = _NC * _NS
_H = 64
_B = 16384
_N = 64
_BPW = _B // _NW
_MARGIN = 1.0
_BLK = 2048


_CHUNK = 16
_GBLK = 128


def _row_gather_blocked(table_hbm, idx_smem, buf_v, out_hbm, out_base, sem,
                        n_rows):
    def blk_body(b):
        def chunk_body(c):
            iv = idx_smem[pl.ds(b * _GBLK + c * _CHUNK, _CHUNK)]
            copies = []
            for jj in range(_CHUNK):
                copies.append(pltpu.async_copy(
                    table_hbm.at[pl.ds(iv[jj], 1), :],
                    buf_v.at[pl.ds(c * _CHUNK + jj, 1), :], sem))
            for cp in copies:
                cp.wait()

        pl.loop(0, _GBLK // _CHUNK)(chunk_body)
        pltpu.sync_copy(buf_v, out_hbm.at[pl.ds(out_base + b * _GBLK, _GBLK)])

    pl.loop(0, n_rows // _GBLK)(blk_body)


def _row_gather_small(table_hbm, idx_smem, buf_v, sem, n_rows):
    for c in range(n_rows // _CHUNK):
        iv = idx_smem[pl.ds(c * _CHUNK, _CHUNK)]
        copies = []
        for jj in range(_CHUNK):
            copies.append(pltpu.async_copy(
                table_hbm.at[pl.ds(iv[jj], 1), :],
                buf_v.at[pl.ds(c * _CHUNK + jj, 1), :], sem))
        for cp in copies:
            cp.wait()


def _sc_gather(emb, r_emb, pos_h, pos_r, pos_t, neg_h, neg_t, neg_r):
    mesh = plsc.VectorSubcoreMesh(core_axis_name="c", subcore_axis_name="s")
    out_type = (
        [jax.ShapeDtypeStruct((_B, _H), jnp.float32)] * 3
        + [jax.ShapeDtypeStruct((_N, _H), jnp.float32)] * 3
    )
    scratch_types = [
        pltpu.VMEM((_BPW,), jnp.int32),
        pltpu.VMEM((_BPW,), jnp.int32),
        pltpu.VMEM((_BPW,), jnp.int32),
        pltpu.VMEM((_GBLK, _H), jnp.float32),
        pltpu.VMEM((_GBLK, _H), jnp.float32),
        pltpu.VMEM((_GBLK, _H), jnp.float32),
        pltpu.VMEM((_N,), jnp.int32),
        pltpu.VMEM((_N, _H), jnp.float32),
        pltpu.SemaphoreType.DMA,
        pltpu.SemaphoreType.DMA,
        pltpu.SemaphoreType.DMA,
        pltpu.SemaphoreType.DMA,
    ]

    @functools.partial(pl.kernel, mesh=mesh, out_type=out_type,
                       scratch_types=scratch_types,
                       compiler_params=pltpu.CompilerParams(
                           use_tc_tiling_on_sc=True))
    def k(emb_hbm, remb_hbm, ph_hbm, pr_hbm, pt_hbm, nh_hbm, nt_hbm, nr_hbm,
          oh, orr, ot, onh, ont, onr,
          ih_s, ir_s, it_s, rh_v, rr_v, rt_v, ni_s, nrow_v,
          sem0, sem1, sem2, sem3):
        wid = lax.axis_index("s") * _NC + lax.axis_index("c")
        base = wid * _BPW
        sl = pl.ds(base, _BPW)
        pltpu.sync_copy(ph_hbm.at[sl], ih_s)
        pltpu.sync_copy(pr_hbm.at[sl], ir_s)
        pltpu.sync_copy(pt_hbm.at[sl], it_s)
        _row_gather_blocked(emb_hbm, ih_s, rh_v, oh, base, sem0, _BPW)
        _row_gather_blocked(remb_hbm, ir_s, rr_v, orr, base, sem1, _BPW)
        _row_gather_blocked(emb_hbm, it_s, rt_v, ot, base, sem2, _BPW)

        @pl.when(wid == 0)
        def _():
            pltpu.sync_copy(nh_hbm, ni_s)
            _row_gather_small(emb_hbm, ni_s, nrow_v, sem0, _N)
            pltpu.sync_copy(nrow_v, onh)

        @pl.when(wid == 1)
        def _():
            pltpu.sync_copy(nt_hbm, ni_s)
            _row_gather_small(emb_hbm, ni_s, nrow_v, sem1, _N)
            pltpu.sync_copy(nrow_v, ont)

        @pl.when(wid == 2)
        def _():
            pltpu.sync_copy(nr_hbm, ni_s)
            _row_gather_small(remb_hbm, ni_s, nrow_v, sem2, _N)
            pltpu.sync_copy(nrow_v, onr)

    return k(emb, r_emb, pos_h, pos_r, pos_t, neg_h, neg_t, neg_r)


_LOG2E = 1.4426950408889634
_LN2 = 0.6931471805599453


def _softplus_neg_sum(x):
    return jnp.sum(jnp.log2(1.0 + jnp.exp2(x * _LOG2E)))


def _softplus_stable(x):
    return jnp.maximum(x, 0.0) + _LN2 * jnp.log2(
        1.0 + jnp.exp2(-jnp.abs(x) * _LOG2E))


def _tc_body(h_ref, r_ref, t_ref, nh_ref, nt_ref, nr_ref, o_ref):
    i = pl.program_id(0)

    @pl.when(i == 0)
    def _():
        o_ref[...] = jnp.zeros((1, 1), jnp.float32)

    h = h_ref[...]
    r = r_ref[...]
    t = t_ref[...]
    nh = nh_ref[...]
    nt = nt_ref[...]
    nr = nr_ref[...]

    d = h + r - t
    rt = r - t
    hr = h + r
    ht = h - t

    def dot_t(a, b):
        return lax.dot_general(a.astype(jnp.bfloat16), b.astype(jnp.bfloat16),
                               (((1,), (1,)), ((), ())),
                               preferred_element_type=jnp.float32)

    nh2 = 0.5 * jnp.sum(nh * nh, axis=1)
    nt2 = 0.5 * jnp.sum(nt * nt, axis=1)
    nr2 = 0.5 * jnp.sum(nr * nr, axis=1)

    pos = _MARGIN - 0.5 * jnp.sum(d * d, axis=1)
    pos_sum = jnp.sum(_softplus_stable(-pos))

    y_nh = (_MARGIN - 0.5 * jnp.sum(rt * rt, axis=1, keepdims=True)
            - nh2[None, :] - dot_t(rt, nh))
    y_nt = (_MARGIN - 0.5 * jnp.sum(hr * hr, axis=1, keepdims=True)
            - nt2[None, :] + dot_t(hr, nt))
    y_nr = (_MARGIN - 0.5 * jnp.sum(ht * ht, axis=1, keepdims=True)
            - nr2[None, :] - dot_t(ht, nr))

    part = (3.0 * pos_sum
            + _LN2 * (_softplus_neg_sum(y_nh)
                      + _softplus_neg_sum(y_nt)
                      + _softplus_neg_sum(y_nr)))
    o_ref[...] += jnp.full((1, 1), part * (1.0 / _B), jnp.float32)


def _tc_score(h_e, r_e, t_e, nh_e, nt_e, nr_e):
    grid = _B // _BLK
    out = pl.pallas_call(
        _tc_body,
        grid=(grid,),
        in_specs=(
            [pl.BlockSpec((_BLK, _H), lambda i: (i, 0))] * 3
            + [pl.BlockSpec((_N, _H), lambda i: (0, 0))] * 3
        ),
        out_specs=pl.BlockSpec((1, 1), lambda i: (0, 0)),
        out_shape=jax.ShapeDtypeStruct((1, 1), jnp.float32),
    )(h_e, r_e, t_e, nh_e, nt_e, nr_e)
    return out[0, 0]


def kernel(pos_h, pos_r, pos_t, neg_h, neg_t, neg_r, emb, r_emb):
    pos_h = pos_h.astype(jnp.int32)
    pos_r = pos_r.astype(jnp.int32)
    pos_t = pos_t.astype(jnp.int32)
    neg_h = neg_h.astype(jnp.int32)
    neg_t = neg_t.astype(jnp.int32)
    neg_r = neg_r.astype(jnp.int32)
    h_e, r_e, t_e, nh_e, nt_e, nr_e = _sc_gather(
        emb, r_emb, pos_h, pos_r, pos_t, neg_h, neg_t, neg_r)
    return _tc_score(h_e, r_e, t_e, nh_e, nt_e, nr_e)
